# baseline (speedup 1.0000x reference)
import jax
import jax.numpy as jnp
from jax.experimental import pallas as pl
from jax.experimental.pallas import tpu as pltpu

_N_NODES = 11
_ROWS_PER_BLOCK = 1024
_COLS = 1024


def _tc_body(c_ref, x_ref, o_ref):
    t = jnp.minimum(jnp.maximum(x_ref[...] * 10.0, 0.0), 10.0)
    acc = jnp.full(t.shape, c_ref[0], t.dtype)
    for i in range(_N_NODES - 1):
        seg = jnp.minimum(jnp.maximum(t - float(i), 0.0), 1.0)
        acc = acc + c_ref[i + 1] * seg
    o_ref[...] = acc


def kernel(input, value):
    n = input.shape[0]
    rows = n // _COLS
    x2 = input.reshape(rows, _COLS)
    coef = jnp.concatenate([value[:1], value[1:] - value[:-1]])
    grid = rows // _ROWS_PER_BLOCK
    out = pl.pallas_call(
        _tc_body,
        grid=(grid,),
        in_specs=[
            pl.BlockSpec(memory_space=pltpu.SMEM),
            pl.BlockSpec((_ROWS_PER_BLOCK, _COLS), lambda i: (i, 0)),
        ],
        out_specs=pl.BlockSpec((_ROWS_PER_BLOCK, _COLS), lambda i: (i, 0)),
        out_shape=jax.ShapeDtypeStruct((rows, _COLS), jnp.float32),
    )(coef, x2)
    return out.reshape(n)
